# baseline (speedup 1.0000x reference)
import jax
import jax.numpy as jnp
from jax.experimental import pallas as pl
from jax.experimental.pallas import tpu as pltpu

N_EXPERTS = 64
TOPK = 8
BLOCK_ROWS = 2048


def _gate_kernel(x_ref, wt_ref, b_ref, w_out_ref, i_out_ref):
    s = jnp.dot(x_ref[...], wt_ref[...], preferred_element_type=jnp.float32)
    s = s + b_ref[...]
    iota = jax.lax.broadcasted_iota(jnp.int32, s.shape, 1)
    b = jax.lax.bitcast_convert_type(s, jnp.int32)
    low = jnp.where(b >= 0, (N_EXPERTS - 1) - iota, iota)
    key = jax.lax.bitcast_convert_type((b & ~(N_EXPERTS - 1)) | low, jnp.float32)
    ms = []
    for k in range(TOPK):
        m = jnp.max(key, axis=1, keepdims=True)
        ms.append(m)
        if k < TOPK - 1:
            key = jnp.where(key == m, -jnp.inf, key)
    vm = jnp.concatenate(ms, axis=1)
    bm = jax.lax.bitcast_convert_type(vm, jnp.int32)
    low6 = bm & (N_EXPERTS - 1)
    i_out_ref[...] = jnp.where(bm >= 0, (N_EXPERTS - 1) - low6, low6)
    v = jax.lax.bitcast_convert_type(bm & ~(N_EXPERTS - 1), jnp.float32)
    e = jnp.exp(v - v[:, 0:1])
    w_out_ref[...] = e / jnp.sum(e, axis=1, keepdims=True)


def kernel(x, weight, bias, target_dist):
    del target_dist
    n_tokens, dim = x.shape
    wt = weight.T
    b2 = bias.reshape(1, N_EXPERTS)
    grid = (n_tokens // BLOCK_ROWS,)
    w_out, i_out = pl.pallas_call(
        _gate_kernel,
        grid=grid,
        in_specs=[
            pl.BlockSpec((BLOCK_ROWS, dim), lambda i: (i, 0)),
            pl.BlockSpec((dim, N_EXPERTS), lambda i: (0, 0)),
            pl.BlockSpec((1, N_EXPERTS), lambda i: (0, 0)),
        ],
        out_specs=[
            pl.BlockSpec((BLOCK_ROWS, TOPK), lambda i: (i, 0)),
            pl.BlockSpec((BLOCK_ROWS, TOPK), lambda i: (i, 0)),
        ],
        out_shape=[
            jax.ShapeDtypeStruct((n_tokens, TOPK), jnp.float32),
            jax.ShapeDtypeStruct((n_tokens, TOPK), jnp.int32),
        ],
    )(x, wt, b2)
    return (w_out, i_out)


# speedup vs baseline: 1.0332x; 1.0332x over previous
import jax
import jax.numpy as jnp
from jax.experimental import pallas as pl
from jax.experimental.pallas import tpu as pltpu

N_EXPERTS = 64
TOPK = 8
BLOCK_ROWS = 1024


def _gate_kernel(x_ref, wt_ref, b_ref, w_out_ref, i_out_ref):
    s = jnp.dot(x_ref[...], wt_ref[...], preferred_element_type=jnp.float32)
    s = s + b_ref[...]
    iota = jax.lax.broadcasted_iota(jnp.int32, s.shape, 1)
    b = jax.lax.bitcast_convert_type(s, jnp.int32)
    low = jnp.where(b >= 0, (N_EXPERTS - 1) - iota, iota)
    key = jax.lax.bitcast_convert_type((b & ~(N_EXPERTS - 1)) | low, jnp.float32)
    ms = []
    for k in range(TOPK):
        m = jnp.max(key, axis=1, keepdims=True)
        ms.append(m)
        if k < TOPK - 1:
            key = jnp.where(key == m, -jnp.inf, key)
    vm = jnp.concatenate(ms, axis=1)
    bm = jax.lax.bitcast_convert_type(vm, jnp.int32)
    low6 = bm & (N_EXPERTS - 1)
    i_out_ref[...] = jnp.where(bm >= 0, (N_EXPERTS - 1) - low6, low6)
    v = jax.lax.bitcast_convert_type(bm & ~(N_EXPERTS - 1), jnp.float32)
    e = jnp.exp(v - v[:, 0:1])
    w_out_ref[...] = e / jnp.sum(e, axis=1, keepdims=True)


def kernel(x, weight, bias, target_dist):
    del target_dist
    n_tokens, dim = x.shape
    wt = weight.T
    b2 = bias.reshape(1, N_EXPERTS)
    grid = (n_tokens // BLOCK_ROWS,)
    w_out, i_out = pl.pallas_call(
        _gate_kernel,
        grid=grid,
        in_specs=[
            pl.BlockSpec((BLOCK_ROWS, dim), lambda i: (i, 0)),
            pl.BlockSpec((dim, N_EXPERTS), lambda i: (0, 0)),
            pl.BlockSpec((1, N_EXPERTS), lambda i: (0, 0)),
        ],
        out_specs=[
            pl.BlockSpec((BLOCK_ROWS, TOPK), lambda i: (i, 0)),
            pl.BlockSpec((BLOCK_ROWS, TOPK), lambda i: (i, 0)),
        ],
        out_shape=[
            jax.ShapeDtypeStruct((n_tokens, TOPK), jnp.float32),
            jax.ShapeDtypeStruct((n_tokens, TOPK), jnp.int32),
        ],
    )(x, wt, b2)
    return (w_out, i_out)
